# baseline (speedup 1.0000x reference)
import jax
import jax.numpy as jnp
from jax import lax
from jax.experimental import pallas as pl
from jax.experimental.pallas import tpu as pltpu
from jax.experimental.pallas import tpu_sc as plsc

DIM = 1024
HID = 512
E = 8
G = 4
NE = 9
SCALE = 1.0
T = 2048
NEG_INF = float("-inf")

NC = 2
NS = 16
NW = NC * NS
TW = T // NW
L = 16


def _dot_nt(a, b):
    return jax.lax.dot_general(a, b, (((1,), (1,)), ((), ())),
                               preferred_element_type=jnp.float32)


def _prep_body(x_ref, wg_ref, bg_ref, bias_ref, s_ref, sb_ref):
    logits = _dot_nt(x_ref[...], wg_ref[...]) + bg_ref[...]
    m = jnp.max(logits, axis=1, keepdims=True)
    ex = jnp.exp(logits - m)
    scores = ex / jnp.sum(ex, axis=1, keepdims=True)
    s_ref[...] = scores
    sb_ref[...] = scores + bias_ref[...]


def _perm(v, idx):
    dnums = lax.GatherDimensionNumbers(
        offset_dims=(), collapsed_slice_dims=(0,), start_index_map=(0,))
    return lax.gather(v, idx.reshape(L, 1), dnums, (1,),
                      mode=lax.GatherScatterMode.PROMISE_IN_BOUNDS)


def _route_body(s_hbm, sb_hbm, comb_hbm, s_v, sb_v, comb_v):
    c = lax.axis_index("c")
    s = lax.axis_index("s")
    wid = s * NC + c
    base = wid * TW * E
    pltpu.sync_copy(s_hbm.at[pl.ds(base, TW * E)], s_v)
    pltpu.sync_copy(sb_hbm.at[pl.ds(base, TW * E)], sb_v)
    neg = jnp.full((L,), NEG_INF, dtype=jnp.float32)
    zero = jnp.zeros((L,), dtype=jnp.float32)
    iota = lax.iota(jnp.int32, L)
    elane = iota & 7
    glane = elane >> 1

    def bmax(v):
        v = jnp.maximum(v, _perm(v, iota ^ 1))
        v = jnp.maximum(v, _perm(v, iota ^ 2))
        return jnp.maximum(v, _perm(v, iota ^ 4))

    def bmin(v):
        v = jnp.minimum(v, _perm(v, iota ^ 1))
        v = jnp.minimum(v, _perm(v, iota ^ 2))
        return jnp.minimum(v, _perm(v, iota ^ 4))

    for k in range(TW * E // L):
        sl = pl.ds(k * L, L)
        sc = s_v[sl]
        sb = sb_v[sl]
        ge = sb + _perm(sb, iota ^ 1)
        gm1 = bmax(ge)
        g1 = bmin(jnp.where(ge == gm1, glane, G))
        ge2 = jnp.where(glane == g1, neg, ge)
        gm2 = bmax(ge2)
        g2 = bmin(jnp.where(ge2 == gm2, glane, G))
        msb = jnp.where((glane == g1) | (glane == g2), sb, neg)
        em1 = bmax(msb)
        e1 = bmin(jnp.where(msb == em1, elane, E))
        msb2 = jnp.where(elane == e1, neg, msb)
        em2 = bmax(msb2)
        e2 = bmin(jnp.where(msb2 == em2, elane, E))
        comb_v[sl] = jnp.where((elane == e1) | (elane == e2),
                               sc * SCALE, zero)
    pltpu.sync_copy(comb_v, comb_hbm.at[pl.ds(base, TW * E)])


def _route_sc(sflat, sbflat):
    mesh = plsc.VectorSubcoreMesh(core_axis_name="c", subcore_axis_name="s",
                                  num_cores=NC, num_subcores=NS)
    f = pl.kernel(
        _route_body,
        out_type=jax.ShapeDtypeStruct((T * E,), jnp.float32),
        mesh=mesh,
        scratch_types=[
            pltpu.VMEM((TW * E,), jnp.float32),
            pltpu.VMEM((TW * E,), jnp.float32),
            pltpu.VMEM((TW * E,), jnp.float32),
        ],
    )
    return f(sflat, sbflat)


def _swiglu(xb, uwm, ubm, gwm, gbm, dwm, dbm):
    b16 = jnp.bfloat16
    xb = xb.astype(b16)
    u = (_dot_nt(xb, uwm.astype(b16)) + ubm).astype(b16)
    g = (_dot_nt(xb, gwm.astype(b16)) + gbm).astype(b16)
    h = u * (1.0 / (1.0 + jnp.exp(-u.astype(jnp.float32)).astype(b16))) * g
    return _dot_nt(h, dwm.astype(b16)) + dbm


def _shared_body(x_ref, suw_ref, sub_ref, sgw_ref, sgb_ref, sdw_ref,
                 sdb_ref, ys_ref):
    ys_ref[...] = _swiglu(x_ref[...], suw_ref[...], sub_ref[...],
                          sgw_ref[...], sgb_ref[...], sdw_ref[...],
                          sdb_ref[...])


def _moe_body(comb_ref, x_ref, ys_ref, uw_ref, ub_ref, gw_ref, gb_ref,
              dw_ref, db_ref, out_ref):
    e = pl.program_id(1)
    col = jax.lax.broadcasted_iota(jnp.int32, comb_ref.shape, 1)
    w = jnp.sum(jnp.where(col == e, comb_ref[...], 0.0), axis=1,
                keepdims=True)
    res = _swiglu(x_ref[...], uw_ref[0], ub_ref[0], gw_ref[0], gb_ref[0],
                  dw_ref[0], db_ref[0]) * w
    out_ref[...] = jnp.where(e == 0, ys_ref[...] + res, out_ref[...] + res)


def kernel(x, Wg, bg, bias, up_w, up_b, gate_w, gate_b, down_w, down_b,
           s_up_w, s_up_b, s_gate_w, s_gate_b, s_down_w, s_down_b):
    orig_shape = x.shape
    x2 = x.reshape(-1, DIM)
    BT = T
    nt = T // BT

    ub = up_b.reshape(E, 1, HID)
    gb = gate_b.reshape(E, 1, HID)
    db = down_b.reshape(E, 1, DIM)
    bg2 = bg.reshape(1, E)
    bias2 = bias.reshape(1, E)
    sub = s_up_b.reshape(1, HID)
    sgb = s_gate_b.reshape(1, HID)
    sdb = s_down_b.reshape(1, DIM)

    st, sbt = pl.pallas_call(
        _prep_body,
        grid=(1,),
        in_specs=[
            pl.BlockSpec((T, DIM), lambda i: (0, 0)),
            pl.BlockSpec((E, DIM), lambda i: (0, 0)),
            pl.BlockSpec((1, E), lambda i: (0, 0)),
            pl.BlockSpec((1, E), lambda i: (0, 0)),
        ],
        out_specs=[pl.BlockSpec((T, E), lambda i: (0, 0)),
                   pl.BlockSpec((T, E), lambda i: (0, 0))],
        out_shape=[jax.ShapeDtypeStruct((T, E), jnp.float32),
                   jax.ShapeDtypeStruct((T, E), jnp.float32)],
    )(x2, Wg, bg2, bias2)

    comb = _route_sc(st.reshape(T * E), sbt.reshape(T * E)).reshape(T, E)
    ys = pl.pallas_call(
        _shared_body,
        grid=(1,),
        in_specs=[
            pl.BlockSpec((T, DIM), lambda i: (0, 0)),
            pl.BlockSpec((HID, DIM), lambda i: (0, 0)),
            pl.BlockSpec((1, HID), lambda i: (0, 0)),
            pl.BlockSpec((HID, DIM), lambda i: (0, 0)),
            pl.BlockSpec((1, HID), lambda i: (0, 0)),
            pl.BlockSpec((DIM, HID), lambda i: (0, 0)),
            pl.BlockSpec((1, DIM), lambda i: (0, 0)),
        ],
        out_specs=pl.BlockSpec((T, DIM), lambda i: (0, 0)),
        out_shape=jax.ShapeDtypeStruct((T, DIM), jnp.float32),
    )(x2, s_up_w, sub, s_gate_w, sgb, s_down_w, sdb)

    out = pl.pallas_call(
        _moe_body,
        grid=(nt, E),
        in_specs=[
            pl.BlockSpec((BT, E), lambda t, e: (t, 0)),
            pl.BlockSpec((BT, DIM), lambda t, e: (t, 0)),
            pl.BlockSpec((BT, DIM), lambda t, e: (t, 0)),
            pl.BlockSpec((1, HID, DIM), lambda t, e: (e, 0, 0)),
            pl.BlockSpec((1, 1, HID), lambda t, e: (e, 0, 0)),
            pl.BlockSpec((1, HID, DIM), lambda t, e: (e, 0, 0)),
            pl.BlockSpec((1, 1, HID), lambda t, e: (e, 0, 0)),
            pl.BlockSpec((1, DIM, HID), lambda t, e: (e, 0, 0)),
            pl.BlockSpec((1, 1, DIM), lambda t, e: (e, 0, 0)),
        ],
        out_specs=pl.BlockSpec((BT, DIM), lambda t, e: (t, 0)),
        out_shape=jax.ShapeDtypeStruct((T, DIM), jnp.float32),
        compiler_params=pltpu.CompilerParams(
            dimension_semantics=("parallel", "arbitrary")),
    )(comb, x2, ys, up_w, ub, gate_w, gb, down_w, db)
    return out.reshape(orig_shape)


# speedup vs baseline: 1.0877x; 1.0877x over previous
import jax
import jax.numpy as jnp
from jax import lax
from jax.experimental import pallas as pl
from jax.experimental.pallas import tpu as pltpu
from jax.experimental.pallas import tpu_sc as plsc

DIM = 1024
HID = 512
E = 8
G = 4
NE = 9
SCALE = 1.0
T = 2048
NEG_INF = float("-inf")

NC = 2
NS = 16
NW = NC * NS
TW = T // NW
L = 16


def _dot_nt(a, b):
    return jax.lax.dot_general(a, b, (((1,), (1,)), ((), ())),
                               preferred_element_type=jnp.float32)


def _prep_body(x_ref, wg_ref, bg_ref, bias_ref, s_ref, sb_ref):
    logits = _dot_nt(x_ref[...], wg_ref[...]) + bg_ref[...]
    m = jnp.max(logits, axis=1, keepdims=True)
    ex = jnp.exp(logits - m)
    scores = ex / jnp.sum(ex, axis=1, keepdims=True)
    s_ref[...] = scores
    sb_ref[...] = scores + bias_ref[...]


def _perm(v, idx):
    dnums = lax.GatherDimensionNumbers(
        offset_dims=(), collapsed_slice_dims=(0,), start_index_map=(0,))
    return lax.gather(v, idx.reshape(L, 1), dnums, (1,),
                      mode=lax.GatherScatterMode.PROMISE_IN_BOUNDS)


def _route_body(s_hbm, sb_hbm, comb_hbm, s_v, sb_v, comb_v):
    c = lax.axis_index("c")
    s = lax.axis_index("s")
    wid = s * NC + c
    base = wid * TW * E
    pltpu.sync_copy(s_hbm.at[pl.ds(base, TW * E)], s_v)
    pltpu.sync_copy(sb_hbm.at[pl.ds(base, TW * E)], sb_v)
    neg = jnp.full((L,), NEG_INF, dtype=jnp.float32)
    zero = jnp.zeros((L,), dtype=jnp.float32)
    iota = lax.iota(jnp.int32, L)
    elane = iota & 7
    glane = elane >> 1

    def bmax(v):
        v = jnp.maximum(v, _perm(v, iota ^ 1))
        v = jnp.maximum(v, _perm(v, iota ^ 2))
        return jnp.maximum(v, _perm(v, iota ^ 4))

    def bmin(v):
        v = jnp.minimum(v, _perm(v, iota ^ 1))
        v = jnp.minimum(v, _perm(v, iota ^ 2))
        return jnp.minimum(v, _perm(v, iota ^ 4))

    for k in range(TW * E // L):
        sl = pl.ds(k * L, L)
        sc = s_v[sl]
        sb = sb_v[sl]
        ge = sb + _perm(sb, iota ^ 1)
        gm1 = bmax(ge)
        g1 = bmin(jnp.where(ge == gm1, glane, G))
        ge2 = jnp.where(glane == g1, neg, ge)
        gm2 = bmax(ge2)
        g2 = bmin(jnp.where(ge2 == gm2, glane, G))
        msb = jnp.where((glane == g1) | (glane == g2), sb, neg)
        em1 = bmax(msb)
        e1 = bmin(jnp.where(msb == em1, elane, E))
        msb2 = jnp.where(elane == e1, neg, msb)
        em2 = bmax(msb2)
        e2 = bmin(jnp.where(msb2 == em2, elane, E))
        comb_v[sl] = jnp.where((elane == e1) | (elane == e2),
                               sc * SCALE, zero)
    pltpu.sync_copy(comb_v, comb_hbm.at[pl.ds(base, TW * E)])


def _route_sc(sflat, sbflat):
    mesh = plsc.VectorSubcoreMesh(core_axis_name="c", subcore_axis_name="s",
                                  num_cores=NC, num_subcores=NS)
    f = pl.kernel(
        _route_body,
        out_type=jax.ShapeDtypeStruct((T * E,), jnp.float32),
        mesh=mesh,
        scratch_types=[
            pltpu.VMEM((TW * E,), jnp.float32),
            pltpu.VMEM((TW * E,), jnp.float32),
            pltpu.VMEM((TW * E,), jnp.float32),
        ],
    )
    return f(sflat, sbflat)


def _swiglu(xb, uwm, ubm, gwm, gbm, dwm, dbm):
    b16 = jnp.bfloat16
    xb = xb.astype(b16)
    u = _dot_nt(xb, uwm.astype(b16)) + ubm
    g = _dot_nt(xb, gwm.astype(b16)) + gbm
    h = u * (1.0 / (1.0 + jnp.exp(-u))) * g
    return _dot_nt(h.astype(b16), dwm.astype(b16)) + dbm


def _moe_body(comb_ref, x_ref, uw_ref, ub_ref, gw_ref, gb_ref, dw_ref,
              db_ref, suw_ref, sub_ref, sgw_ref, sgb_ref, sdw_ref, sdb_ref,
              out_ref):
    e = pl.program_id(1)

    @pl.when(e < E)
    def _():
        col = jax.lax.broadcasted_iota(jnp.int32, comb_ref.shape, 1)
        w = jnp.sum(jnp.where(col == e, comb_ref[...], 0.0), axis=1,
                    keepdims=True)
        res = _swiglu(x_ref[...], uw_ref[0], ub_ref[0], gw_ref[0], gb_ref[0],
                      dw_ref[0], db_ref[0]) * w
        out_ref[...] = jnp.where(e == 0, res, out_ref[...] + res)

    @pl.when(e == E)
    def _():
        out_ref[...] += _swiglu(x_ref[...], suw_ref[...], sub_ref[...],
                                sgw_ref[...], sgb_ref[...], sdw_ref[...],
                                sdb_ref[...])


def kernel(x, Wg, bg, bias, up_w, up_b, gate_w, gate_b, down_w, down_b,
           s_up_w, s_up_b, s_gate_w, s_gate_b, s_down_w, s_down_b):
    orig_shape = x.shape
    x2 = x.reshape(-1, DIM)
    BT = T
    nt = T // BT

    ub = up_b.reshape(E, 1, HID)
    gb = gate_b.reshape(E, 1, HID)
    db = down_b.reshape(E, 1, DIM)
    bg2 = bg.reshape(1, E)
    bias2 = bias.reshape(1, E)
    sub = s_up_b.reshape(1, HID)
    sgb = s_gate_b.reshape(1, HID)
    sdb = s_down_b.reshape(1, DIM)

    st, sbt = pl.pallas_call(
        _prep_body,
        grid=(1,),
        in_specs=[
            pl.BlockSpec((T, DIM), lambda i: (0, 0)),
            pl.BlockSpec((E, DIM), lambda i: (0, 0)),
            pl.BlockSpec((1, E), lambda i: (0, 0)),
            pl.BlockSpec((1, E), lambda i: (0, 0)),
        ],
        out_specs=[pl.BlockSpec((T, E), lambda i: (0, 0)),
                   pl.BlockSpec((T, E), lambda i: (0, 0))],
        out_shape=[jax.ShapeDtypeStruct((T, E), jnp.float32),
                   jax.ShapeDtypeStruct((T, E), jnp.float32)],
    )(x2, Wg, bg2, bias2)

    comb = _route_sc(st.reshape(T * E), sbt.reshape(T * E)).reshape(T, E)

    ecap = lambda t, e: (jnp.minimum(e, E - 1), 0, 0)
    out = pl.pallas_call(
        _moe_body,
        grid=(nt, NE),
        in_specs=[
            pl.BlockSpec((BT, E), lambda t, e: (t, 0)),
            pl.BlockSpec((BT, DIM), lambda t, e: (t, 0)),
            pl.BlockSpec((1, HID, DIM), ecap),
            pl.BlockSpec((1, 1, HID), ecap),
            pl.BlockSpec((1, HID, DIM), ecap),
            pl.BlockSpec((1, 1, HID), ecap),
            pl.BlockSpec((1, DIM, HID), ecap),
            pl.BlockSpec((1, 1, DIM), ecap),
            pl.BlockSpec((HID, DIM), lambda t, e: (0, 0)),
            pl.BlockSpec((1, HID), lambda t, e: (0, 0)),
            pl.BlockSpec((HID, DIM), lambda t, e: (0, 0)),
            pl.BlockSpec((1, HID), lambda t, e: (0, 0)),
            pl.BlockSpec((DIM, HID), lambda t, e: (0, 0)),
            pl.BlockSpec((1, DIM), lambda t, e: (0, 0)),
        ],
        out_specs=pl.BlockSpec((BT, DIM), lambda t, e: (t, 0)),
        out_shape=jax.ShapeDtypeStruct((T, DIM), jnp.float32),
        compiler_params=pltpu.CompilerParams(
            dimension_semantics=("parallel", "arbitrary")),
    )(comb, x2, up_w, ub, gate_w, gb, down_w, db,
      s_up_w, sub, s_gate_w, sgb, s_down_w, sdb)
    return out.reshape(orig_shape)
